# baseline (speedup 1.0000x reference)
import functools

import jax
import jax.numpy as jnp
from jax import lax
from jax.experimental import pallas as pl
from jax.experimental.pallas import tpu as pltpu
from jax.experimental.pallas import tpu_sc as plsc

IN_SIZE = 1024
HID = 1024
MEM_SIZE = 65536
MEM_DIM = 512
OUT_SIZE = 1024
TOP_K = 3
BLK = 4096
NBLK = MEM_SIZE // BLK
NEG_INF = float("-inf")
IMAX = 2**31 - 1

NTILE = 16
TILE_N = MEM_SIZE // NTILE
LANE = 16


def _tc1_body(xlast_ref, W1_ref, b1_ref, W2_ref, b2_ref, Wq_ref, bq_ref,
              keys_ref, imp_ref, Wout_t_ref, bout_ref,
              wsims_ref, part_ref, qn_s):
    step = pl.program_id(0)

    @pl.when(step == 0)
    def _init():
        x = xlast_ref[...]
        h1 = jnp.maximum(
            jnp.dot(x, W1_ref[...], preferred_element_type=jnp.float32)
            + b1_ref[...], 0.0)
        h2 = jnp.dot(h1, W2_ref[...], preferred_element_type=jnp.float32) \
            + b2_ref[...]
        part_ref[...] = jnp.dot(h2, Wout_t_ref[...],
                                preferred_element_type=jnp.float32) \
            + bout_ref[...]
        q = jnp.dot(h2, Wq_ref[...], preferred_element_type=jnp.float32) \
            + bq_ref[...]
        qnorm = jnp.sqrt(jnp.sum(q * q))
        qn_s[...] = q / jnp.maximum(qnorm, 1e-12)

    blk = keys_ref[...]
    qn = qn_s[...]
    dn = (((1,), (1,)), ((), ()))
    dots = lax.dot_general(qn, blk, dn,
                           preferred_element_type=jnp.float32)
    sq = blk * blk
    ones = jnp.ones((1, MEM_DIM), dtype=jnp.float32)
    rn = lax.dot_general(ones, sq, dn,
                         preferred_element_type=jnp.float32)
    w = dots / jnp.maximum(jnp.sqrt(rn), 1e-12) * imp_ref[0]
    wsims_ref[...] = w.reshape(1, 1, BLK)


def _tc1(x_last, W1, b1, W2, b2, Wq, bq, mem_keys, imp3, Wout_top, bout):
    full = lambda i: (0, 0)
    grid_spec = pltpu.PrefetchScalarGridSpec(
        num_scalar_prefetch=0,
        grid=(NBLK,),
        in_specs=[
            pl.BlockSpec((1, IN_SIZE), full),
            pl.BlockSpec((IN_SIZE, HID), full),
            pl.BlockSpec((1, HID), full),
            pl.BlockSpec((HID, HID), full),
            pl.BlockSpec((1, HID), full),
            pl.BlockSpec((HID, MEM_DIM), full),
            pl.BlockSpec((1, MEM_DIM), full),
            pl.BlockSpec((BLK, MEM_DIM), lambda i: (i, 0)),
            pl.BlockSpec((1, 1, BLK), lambda i: (i, 0, 0)),
            pl.BlockSpec((HID, OUT_SIZE), full),
            pl.BlockSpec((1, OUT_SIZE), full),
        ],
        out_specs=(
            pl.BlockSpec((1, 1, BLK), lambda i: (i, 0, 0)),
            pl.BlockSpec((1, OUT_SIZE), full),
        ),
        scratch_shapes=[pltpu.VMEM((1, MEM_DIM), jnp.float32)],
    )
    return pl.pallas_call(
        _tc1_body,
        grid_spec=grid_spec,
        out_shape=(
            jax.ShapeDtypeStruct((NBLK, 1, BLK), jnp.float32),
            jax.ShapeDtypeStruct((1, OUT_SIZE), jnp.float32),
        ),
        compiler_params=pltpu.CompilerParams(
            dimension_semantics=("arbitrary",),
        ),
    )(x_last, W1, b1, W2, b2, Wq, bq, mem_keys, imp3, Wout_top, bout)


def _iota16():
    return lax.broadcasted_iota(jnp.int32, (LANE,), 0)


def _insert_topk(v, i, tv, ti):
    v1, v2, v3 = tv
    i1, i2, i3 = ti
    c1 = v > v1
    c2 = v > v2
    c3 = v > v3
    n3 = jnp.where(c2, v2, jnp.where(c3, v, v3))
    j3 = jnp.where(c2, i2, jnp.where(c3, i, i3))
    n2 = jnp.where(c1, v1, jnp.where(c2, v, v2))
    j2 = jnp.where(c1, i1, jnp.where(c2, i, i2))
    n1 = jnp.where(c1, v, v1)
    j1 = jnp.where(c1, i, i1)
    return (n1, n2, n3), (j1, j2, j3)


def _take16(v, idx):
    dn = lax.GatherDimensionNumbers(
        offset_dims=(), collapsed_slice_dims=(0,), start_index_map=(0,))
    return lax.gather(v, idx[:, None], dn, slice_sizes=(1,),
                      mode=lax.GatherScatterMode.PROMISE_IN_BOUNDS)


def _butterfly(v, op):
    it = _iota16()
    for k in (1, 2, 4, 8):
        v = op(v, _take16(v, jnp.bitwise_xor(it, k)))
    return v


def _bcast_max(v):
    return _butterfly(v, jnp.maximum)


def _bcast_min(v):
    return _butterfly(v, jnp.minimum)


def _bcast_sum(v):
    return _butterfly(v, jnp.add)


def _extract_max(tv, ti):
    v1, v2, v3 = tv
    i1, i2, i3 = ti
    g = _bcast_max(v1)
    eq = v1 == g
    sel = _bcast_min(jnp.where(eq, i1, IMAX))
    rem = eq & (i1 == sel)
    v1 = jnp.where(rem, v2, v1)
    i1 = jnp.where(rem, i2, i1)
    v2 = jnp.where(rem, v3, v2)
    i2 = jnp.where(rem, i3, i2)
    v3 = jnp.where(rem, NEG_INF, v3)
    return g, sel, (v1, v2, v3), (i1, i2, i3)


def _splats_to_vec(splats, fill, dtype):
    vec = jnp.full((LANE,), fill, dtype=dtype)
    it = _iota16()
    for j, s in enumerate(splats):
        vec = jnp.where(it == j, s, vec)
    return vec


NW = 32
TILE32 = MEM_SIZE // NW


def _sc_scan_body(wsims_hbm, vals_hbm, idxs_hbm, sims_v, triple_v, triple_i):
    cid = lax.axis_index("c")
    sid = lax.axis_index("s")
    wid = sid * 2 + cid
    base = wid * TILE32
    pltpu.sync_copy(wsims_hbm.at[pl.ds(base, TILE32)], sims_v)
    it = _iota16()

    def scan_step(k, carry):
        tv = carry[0:3]
        ti = carry[3:6]
        v = sims_v[pl.ds(k * LANE, LANE)]
        idx = base + k * LANE + it
        tv, ti = _insert_topk(v, idx, tv, ti)
        return tv + ti

    ninf = jnp.full((LANE,), NEG_INF, dtype=jnp.float32)
    zero = jnp.zeros((LANE,), dtype=jnp.int32)
    carry = lax.fori_loop(0, TILE32 // LANE, scan_step,
                          (ninf, ninf, ninf, zero, zero, zero))
    tv = carry[0:3]
    ti = carry[3:6]
    vals, idxs = [], []
    for _ in range(TOP_K):
        g, sel, tv, ti = _extract_max(tv, ti)
        vals.append(g)
        idxs.append(sel)
    triple_v[...] = _splats_to_vec(vals, NEG_INF, jnp.float32)
    triple_i[...] = _splats_to_vec(idxs, 0, jnp.int32)
    pltpu.sync_copy(triple_v, vals_hbm.at[wid])
    pltpu.sync_copy(triple_i, idxs_hbm.at[wid])


@functools.partial(
    pl.kernel,
    mesh=plsc.VectorSubcoreMesh(core_axis_name="c", subcore_axis_name="s"),
    out_type=(
        jax.ShapeDtypeStruct((NW, LANE), jnp.float32),
        jax.ShapeDtypeStruct((NW, LANE), jnp.int32),
    ),
    scratch_types=[
        pltpu.VMEM((TILE32,), jnp.float32),
        pltpu.VMEM((LANE,), jnp.float32),
        pltpu.VMEM((LANE,), jnp.int32),
    ],
)
def _sc_scan(wsims_hbm, vals_hbm, idxs_hbm, *scratch):
    _sc_scan_body(wsims_hbm, vals_hbm, idxs_hbm, *scratch)


def _sc_merge_body(vals_hbm, idxs_hbm, mem_vals_hbm, out_hbm,
                   allv_v, alli_v, idx_v, rows_v, out_v, sem):
    cid = lax.axis_index("c")
    sid = lax.axis_index("s")

    @pl.when((cid == 0) & (sid == 0))
    def _():
        pltpu.sync_copy(vals_hbm, allv_v)
        pltpu.sync_copy(idxs_hbm, alli_v)
        it = _iota16()
        tv = (jnp.full((LANE,), NEG_INF, dtype=jnp.float32),) * 3
        ti = (jnp.zeros((LANE,), dtype=jnp.int32),) * 3
        for r in range(NW):
            tv, ti = _insert_topk(allv_v[r], alli_v[r], tv, ti)
        gvals, gidxs = [], []
        for _ in range(TOP_K):
            g, sel, tv, ti = _extract_max(tv, ti)
            gvals.append(g)
            gidxs.append(sel)

        idx_v[...] = _splats_to_vec(gidxs, 0, jnp.int32)
        pltpu.async_copy(mem_vals_hbm.at[idx_v], rows_v, sem).wait()

        m0 = gvals[0]
        sv = _splats_to_vec(gvals, NEG_INF, jnp.float32)
        ev = jnp.exp(sv - m0)
        den = _bcast_sum(ev)
        ninfv = jnp.full((LANE,), NEG_INF, dtype=jnp.float32)
        attn = [_bcast_max(jnp.where(it == j, ev, ninfv)) / den
                for j in range(TOP_K)]
        for c in range(MEM_DIM // LANE):
            sl = pl.ds(c * LANE, LANE)
            acc = (attn[0] * rows_v[0, sl] + attn[1] * rows_v[1, sl]
                   + attn[2] * rows_v[2, sl])
            out_v[sl] = acc
        pltpu.sync_copy(out_v, out_hbm)


@functools.partial(
    pl.kernel,
    mesh=plsc.VectorSubcoreMesh(core_axis_name="c", subcore_axis_name="s"),
    out_type=jax.ShapeDtypeStruct((MEM_DIM,), jnp.float32),
    scratch_types=[
        pltpu.VMEM((NW, LANE), jnp.float32),
        pltpu.VMEM((NW, LANE), jnp.int32),
        pltpu.VMEM((LANE,), jnp.int32),
        pltpu.VMEM((LANE, MEM_DIM), jnp.float32),
        pltpu.VMEM((MEM_DIM,), jnp.float32),
        pltpu.SemaphoreType.DMA,
    ],
)
def _sc_merge(vals_hbm, idxs_hbm, mem_vals_hbm, out_hbm, *scratch):
    _sc_merge_body(vals_hbm, idxs_hbm, mem_vals_hbm, out_hbm, *scratch)


def _sc_retrieve(wsims, mem_vals):
    vals, idxs = _sc_scan(wsims)
    return _sc_merge(vals, idxs, mem_vals)


def _tc2_body(part_ref, retr_ref, Wout_b_ref, out_ref):
    out_ref[...] = part_ref[...] + jnp.dot(
        retr_ref[...], Wout_b_ref[...], preferred_element_type=jnp.float32)


def _tc2(part, retrieved, Wout_bot):
    return pl.pallas_call(
        _tc2_body,
        out_shape=jax.ShapeDtypeStruct((1, OUT_SIZE), jnp.float32),
    )(part, retrieved, Wout_bot)


def kernel(x, W1, b1, W2, b2, Wq, bq, mem_keys, mem_vals, importance, Wout, bout):
    x_last = x[:, -1, :]
    imp3 = importance.reshape(NBLK, 1, BLK)
    wsims, part = _tc1(x_last, W1, b1.reshape(1, HID), W2, b2.reshape(1, HID),
                       Wq, bq.reshape(1, MEM_DIM), mem_keys, imp3,
                       Wout[:HID], bout.reshape(1, OUT_SIZE))
    retrieved = _sc_retrieve(wsims.reshape(MEM_SIZE), mem_vals)
    return _tc2(part, retrieved.reshape(1, MEM_DIM), Wout[HID:])
